# baseline (speedup 1.0000x reference)
import jax
import jax.numpy as jnp
from jax.experimental import pallas as pl
from jax.experimental.pallas import tpu as pltpu

_N = 1024
_H = 256
_EPS_S = 0.005 ** 2
_SINK_ITERS = 20
_AUCTION_ITERS = 5
_BID_EPS = 1e-3


def _fused_kernel(ct_ref, n_ref, t_ref, w1_ref, b1_ref, w2t_ref, b2_ref,
                  vp_ref, v_ref):
    N = _N
    cloudT = ct_ref[0]
    noise = n_ref[0]
    t = t_ref[0, 0, 0]

    mu = jnp.mean(cloudT)
    std = jnp.sqrt(jnp.mean((cloudT - mu) ** 2))
    x0T = cloudT / std

    d2 = (noise[:, 0:1] - x0T[0:1, :]) ** 2
    d2 = d2 + (noise[:, 1:2] - x0T[1:2, :]) ** 2
    d2 = d2 + (noise[:, 2:3] - x0T[2:3, :]) ** 2

    C = d2 * 0.5
    loga = -jnp.log(jnp.float32(N))
    logb = -jnp.log(jnp.float32(N))

    Ceps = C / _EPS_S
    ones_col = jnp.ones((N, 1), jnp.float32)
    ones_row = jnp.ones((1, N), jnp.float32)
    hi = jax.lax.Precision.HIGHEST

    def sink(_, FG):
        F, G = FG
        A = (G + logb) - Ceps
        m = jnp.max(A, axis=1, keepdims=True)
        s = jax.lax.dot_general(jnp.exp(A - m), ones_col,
                                (((1,), (0,)), ((), ())),
                                precision=hi,
                                preferred_element_type=jnp.float32)
        F = -(jnp.log(s) + m)
        A2 = (F + loga) - Ceps
        m2 = jnp.max(A2, axis=0, keepdims=True)
        s2 = jax.lax.dot_general(ones_row, jnp.exp(A2 - m2),
                                 (((1,), (0,)), ((), ())),
                                 precision=hi,
                                 preferred_element_type=jnp.float32)
        G = -(jnp.log(s2) + m2)
        return (F, G)

    f0 = jnp.zeros((N, 1), jnp.float32)
    g0 = jnp.zeros((1, N), jnp.float32)
    _, G = jax.lax.fori_loop(0, _SINK_ITERS, sink, (f0, g0))
    price = _EPS_S * (-G)

    jcol = jax.lax.broadcasted_iota(jnp.int32, (N, N), 1)
    neg_inf = jnp.float32(-jnp.inf)

    def auct(_, carry):
        price, _best = carry
        score = d2 + price
        s1 = jnp.max(score, axis=1, keepdims=True)
        idx = jnp.min(jnp.where(score == s1, jcol, N), axis=1, keepdims=True)
        hit = jcol == idx
        s2 = jnp.max(jnp.where(hit, neg_inf, score), axis=1, keepdims=True)
        bid = (s2 - s1) + jnp.float32(_BID_EPS)
        scat = jnp.max(jnp.where(hit, bid, neg_inf), axis=0, keepdims=True)
        price = jnp.where(scat != neg_inf, scat, price)
        return (price, idx)

    best0 = jnp.zeros((N, 1), jnp.int32)
    _, best = jax.lax.fori_loop(0, _AUCTION_ITERS, auct, (price, best0))

    hits = (jcol == best).astype(jnp.float32)
    x0_al = jax.lax.dot_general(hits, x0T, (((1,), (1,)), ((), ())),
                                precision=hi,
                                preferred_element_type=jnp.float32)

    x_t = (1.0 - t) * x0_al + t * noise
    v = noise - x0_al

    w1 = w1_ref[...]
    b1 = b1_ref[...]
    w2t = w2t_ref[...]
    b2 = b2_ref[...]
    pre = (x_t[:, 0:1] * w1[0:1, :] + x_t[:, 1:2] * w1[1:2, :]
           + x_t[:, 2:3] * w1[2:3, :] + t * w1[3:4, :] + b1)
    h = jnp.tanh(pre)
    vp = jnp.concatenate(
        [jnp.sum(h * w2t[c:c + 1, :], axis=1, keepdims=True) for c in range(3)],
        axis=1) + b2

    vp_ref[0] = vp
    v_ref[0] = v


def kernel(cloud, noise, t, W1, b1, W2, b2):
    B, N, _ = cloud.shape
    H = W1.shape[1]
    cloudT = jnp.swapaxes(cloud, 1, 2)
    t3 = t.reshape(B, 1, 1)
    b1r = b1.reshape(1, H)
    W2T = W2.T
    b2r = b2.reshape(1, 3)
    vp, v = pl.pallas_call(
        _fused_kernel,
        grid=(B,),
        in_specs=[
            pl.BlockSpec((1, 3, N), lambda b: (b, 0, 0)),
            pl.BlockSpec((1, N, 3), lambda b: (b, 0, 0)),
            pl.BlockSpec((1, 1, 1), lambda b: (b, 0, 0)),
            pl.BlockSpec((4, H), lambda b: (0, 0)),
            pl.BlockSpec((1, H), lambda b: (0, 0)),
            pl.BlockSpec((3, H), lambda b: (0, 0)),
            pl.BlockSpec((1, 3), lambda b: (0, 0)),
        ],
        out_specs=[
            pl.BlockSpec((1, N, 3), lambda b: (b, 0, 0)),
            pl.BlockSpec((1, N, 3), lambda b: (b, 0, 0)),
        ],
        out_shape=[jax.ShapeDtypeStruct((B, N, 3), jnp.float32)] * 2,
        compiler_params=pltpu.CompilerParams(
            dimension_semantics=("parallel",)),
    )(cloudT, noise, t3, W1, b1r, W2T, b2r)
    return (vp, v)


# speedup vs baseline: 3.7555x; 3.7555x over previous
import jax
import jax.numpy as jnp
from jax.experimental import pallas as pl
from jax.experimental.pallas import tpu as pltpu

_N = 1024
_H = 256
_EPS_S = 0.005 ** 2
_SINK_ITERS = 20
_AUCTION_ITERS = 5
_BID_EPS = 1e-3


def _fused_kernel(ct_ref, n_ref, t_ref, w1_ref, b1_ref, w2t_ref, b2_ref,
                  vp_ref, v_ref):
    N = _N
    cloudT = ct_ref[0]
    noise = n_ref[0]
    t = t_ref[0, 0, 0]

    mu = jnp.mean(cloudT)
    std = jnp.sqrt(jnp.mean((cloudT - mu) ** 2))
    x0T = cloudT / std

    d2 = (noise[:, 0:1] - x0T[0:1, :]) ** 2
    d2 = d2 + (noise[:, 1:2] - x0T[1:2, :]) ** 2
    d2 = d2 + (noise[:, 2:3] - x0T[2:3, :]) ** 2

    C = d2 * 0.5
    loga = -jnp.log(jnp.float32(N))
    logb = -jnp.log(jnp.float32(N))

    log2e = jnp.float32(1.4426950408889634)
    Ceps2 = (C / _EPS_S) * log2e
    logb2 = logb * log2e
    loga2 = loga * log2e

    def sink(_, FG):
        F2, G2 = FG
        A = (G2 + logb2) - Ceps2
        m = jnp.max(A, axis=1, keepdims=True)
        F2 = -(jnp.log2(jnp.sum(jnp.exp2(A - m), axis=1, keepdims=True)) + m)
        A2 = (F2 + loga2) - Ceps2
        m2 = jnp.max(A2, axis=0, keepdims=True)
        G2 = -(jnp.log2(jnp.sum(jnp.exp2(A2 - m2), axis=0, keepdims=True)) + m2)
        return (F2, G2)

    f0 = jnp.zeros((N, 1), jnp.float32)
    g0 = jnp.zeros((1, N), jnp.float32)
    _, G2 = jax.lax.fori_loop(0, _SINK_ITERS, sink, (f0, g0))
    price = jnp.float32(_EPS_S * 0.6931471805599453) * (-G2)

    jcol = jax.lax.broadcasted_iota(jnp.int32, (N, N), 1)
    neg_inf = jnp.float32(-jnp.inf)

    def auct(_, carry):
        price, _best = carry
        score = d2 + price
        s1 = jnp.max(score, axis=1, keepdims=True)
        idx = jnp.min(jnp.where(score == s1, jcol, N), axis=1, keepdims=True)
        hit = jcol == idx
        s2 = jnp.max(jnp.where(hit, neg_inf, score), axis=1, keepdims=True)
        bid = (s2 - s1) + jnp.float32(_BID_EPS)
        scat = jnp.max(jnp.where(hit, bid, neg_inf), axis=0, keepdims=True)
        price = jnp.where(scat != neg_inf, scat, price)
        return (price, idx)

    best0 = jnp.zeros((N, 1), jnp.int32)
    _, best = jax.lax.fori_loop(0, _AUCTION_ITERS, auct, (price, best0))

    hits = jcol == best
    cols = [jnp.sum(jnp.where(hits, x0T[k:k + 1, :], 0.0), axis=1, keepdims=True)
            for k in range(3)]
    x0_al = jnp.concatenate(cols, axis=1)

    x_t = (1.0 - t) * x0_al + t * noise
    v = noise - x0_al

    w1 = w1_ref[...]
    b1 = b1_ref[...]
    w2t = w2t_ref[...]
    b2 = b2_ref[...]
    pre = (x_t[:, 0:1] * w1[0:1, :] + x_t[:, 1:2] * w1[1:2, :]
           + x_t[:, 2:3] * w1[2:3, :] + t * w1[3:4, :] + b1)
    h = jnp.tanh(pre)
    vp = jnp.concatenate(
        [jnp.sum(h * w2t[c:c + 1, :], axis=1, keepdims=True) for c in range(3)],
        axis=1) + b2

    vp_ref[0] = vp
    v_ref[0] = v


def kernel(cloud, noise, t, W1, b1, W2, b2):
    B, N, _ = cloud.shape
    H = W1.shape[1]
    cloudT = jnp.swapaxes(cloud, 1, 2)
    t3 = t.reshape(B, 1, 1)
    b1r = b1.reshape(1, H)
    W2T = W2.T
    b2r = b2.reshape(1, 3)
    vp, v = pl.pallas_call(
        _fused_kernel,
        grid=(B,),
        in_specs=[
            pl.BlockSpec((1, 3, N), lambda b: (b, 0, 0)),
            pl.BlockSpec((1, N, 3), lambda b: (b, 0, 0)),
            pl.BlockSpec((1, 1, 1), lambda b: (b, 0, 0)),
            pl.BlockSpec((4, H), lambda b: (0, 0)),
            pl.BlockSpec((1, H), lambda b: (0, 0)),
            pl.BlockSpec((3, H), lambda b: (0, 0)),
            pl.BlockSpec((1, 3), lambda b: (0, 0)),
        ],
        out_specs=[
            pl.BlockSpec((1, N, 3), lambda b: (b, 0, 0)),
            pl.BlockSpec((1, N, 3), lambda b: (b, 0, 0)),
        ],
        out_shape=[jax.ShapeDtypeStruct((B, N, 3), jnp.float32)] * 2,
        compiler_params=pltpu.CompilerParams(
            dimension_semantics=("parallel",)),
    )(cloudT, noise, t3, W1, b1r, W2T, b2r)
    return (vp, v)
